# baseline (speedup 1.0000x reference)
import functools

import jax
import jax.numpy as jnp
from jax import lax
from jax.experimental import pallas as pl
from jax.experimental.pallas import tpu as pltpu
from jax.experimental.pallas import tpu_sc as plsc

BATCH = 4096
HIST = 200
EMB = 64

NC = 2
NS = 16
NW = NC * NS

BW = BATCH // NW
L = 16


def _make_gather():
    mesh = plsc.VectorSubcoreMesh(
        core_axis_name="c", subcore_axis_name="s", num_cores=NC, num_subcores=NS
    )

    @functools.partial(
        pl.kernel,
        mesh=mesh,
        compiler_params=pltpu.CompilerParams(
            use_tc_tiling_on_sc=False, needs_layout_passes=False
        ),
        out_type=jax.ShapeDtypeStruct((HIST, EMB, BATCH), jnp.float32),
        scratch_types=[
            pltpu.VMEM((HIST, BW), jnp.int32),
            pltpu.VMEM((2, BW, EMB), jnp.float32),
            pltpu.VMEM((2, EMB, BW), jnp.float32),
            pltpu.SemaphoreType.DMA,
            pltpu.SemaphoreType.DMA,
            pltpu.SemaphoreType.DMA,
            pltpu.SemaphoreType.DMA,
        ],
    )
    def gather_kernel(xt_hbm, table_hbm, out_hbm, idx_v, g_v, t_v,
                      gsem0, gsem1, osem0, osem1):
        wid = lax.axis_index("s") * NC + lax.axis_index("c")
        wb = wid * BW
        pltpu.sync_copy(xt_hbm.at[:, pl.ds(wb, BW)], idx_v)

        gsems = (gsem0, gsem1)
        osems = (osem0, osem1)

        def start_gather(h, b):
            pltpu.async_copy(table_hbm.at[idx_v.at[h]], g_v.at[b], gsems[b])

        def drain_gather(b):
            pltpu.make_async_copy(
                table_hbm.at[idx_v.at[0]], g_v.at[b], gsems[b]
            ).wait()

        def start_out(h, b):
            pltpu.async_copy(
                t_v.at[b], out_hbm.at[h, :, pl.ds(wb, BW)], osems[b]
            )

        def wait_out(b):
            pltpu.make_async_copy(
                t_v.at[b], out_hbm.at[0, :, pl.ds(wb, BW)], osems[b]
            ).wait()

        tgroups = [lax.iota(jnp.int32, L) + (tg * L) for tg in range(BW // L)]

        def transpose(b):
            for e in range(EMB):
                cols = jnp.full((L,), e, jnp.int32)
                for tg in range(BW // L):
                    vals = plsc.load_gather(g_v.at[b], [tgroups[tg], cols])
                    t_v[b, e, pl.ds(tg * L, L)] = vals

        start_gather(0, 0)

        def outer(ho, _):
            for b in range(2):
                h = ho * 2 + b
                drain_gather(b)
                @pl.when(h + 1 < HIST)
                def _():
                    start_gather(h + 1, 1 - b)
                @pl.when(h >= 2)
                def _():
                    wait_out(b)
                transpose(b)
                start_out(h, b)
            return 0

        lax.fori_loop(0, HIST // 2, outer, 0)
        wait_out(0)
        wait_out(1)

    return gather_kernel


_gather = _make_gather()


def kernel(x, table):
    out_t = _gather(x.astype(jnp.int32).T, table)
    return out_t.transpose(2, 0, 1)


# speedup vs baseline: 1.7908x; 1.7908x over previous
import functools

import jax
import jax.numpy as jnp
from jax import lax
from jax.experimental import pallas as pl
from jax.experimental.pallas import tpu as pltpu
from jax.experimental.pallas import tpu_sc as plsc

BATCH = 4096
HIST = 200
EMB = 64

NC = 2
NS = 16
NW = NC * NS

B = BATCH * HIST
GW = 128
SUB = 4
CHUNK = GW * SUB
B_PER_W = B // NW
NCHUNK = B_PER_W // CHUNK
ROWS_PER_W = B_PER_W // GW


def _make_gather():
    mesh = plsc.VectorSubcoreMesh(
        core_axis_name="c", subcore_axis_name="s", num_cores=NC, num_subcores=NS
    )

    @functools.partial(
        pl.kernel,
        mesh=mesh,
        compiler_params=pltpu.CompilerParams(use_tc_tiling_on_sc=False),
        out_type=jax.ShapeDtypeStruct((B // GW, GW, EMB), jnp.float32),
        scratch_types=[
            pltpu.VMEM((ROWS_PER_W, GW), jnp.int32),
            pltpu.VMEM((2, SUB, GW, EMB), jnp.float32),
            pltpu.SemaphoreType.DMA,
            pltpu.SemaphoreType.DMA,
            pltpu.SemaphoreType.DMA,
            pltpu.SemaphoreType.DMA,
        ],
    )
    def gather_kernel(idx_hbm, table_hbm, out_hbm, idx_v, rows_v,
                      gsem0, gsem1, osem0, osem1):
        wid = lax.axis_index("s") * NC + lax.axis_index("c")
        row_base = wid * ROWS_PER_W
        pltpu.sync_copy(idx_hbm.at[pl.ds(row_base, ROWS_PER_W)], idx_v)

        gsems = (gsem0, gsem1)
        osems = (osem0, osem1)

        def start_gathers(g, b):
            for j in range(SUB):
                pltpu.async_copy(
                    table_hbm.at[idx_v.at[g * SUB + j]],
                    rows_v.at[b, j],
                    gsems[b],
                )

        def drain_gathers(b):
            for j in range(SUB):
                pltpu.make_async_copy(
                    table_hbm.at[idx_v.at[j]], rows_v.at[b, j], gsems[b]
                ).wait()

        def start_out(g, b):
            pltpu.async_copy(
                rows_v.at[b],
                out_hbm.at[pl.ds(row_base + g * SUB, SUB)],
                osems[b],
            )

        def wait_out(b):
            pltpu.make_async_copy(
                rows_v.at[b], out_hbm.at[pl.ds(row_base, SUB)], osems[b]
            ).wait()

        start_gathers(0, 0)

        def outer(go, _):
            @pl.when(go >= 1)
            def _():
                wait_out(1)
            start_gathers(go * 2 + 1, 1)
            drain_gathers(0)
            start_out(go * 2, 0)
            @pl.when(go < NCHUNK // 2 - 1)
            def _():
                wait_out(0)
                start_gathers(go * 2 + 2, 0)
            drain_gathers(1)
            start_out(go * 2 + 1, 1)
            return 0

        lax.fori_loop(0, NCHUNK // 2, outer, 0)
        wait_out(0)
        wait_out(1)

    return gather_kernel


_gather = _make_gather()


def kernel(x, table):
    idx = x.astype(jnp.int32).reshape(B // GW, GW)
    out = _gather(idx, table)
    return out.reshape(BATCH, HIST, EMB)
